# Initial kernel scaffold; baseline (speedup 1.0000x reference)
import jax
import jax.numpy as jnp
from jax.experimental import pallas as pl


def kernel(X, M):
    raise NotImplementedError("write your pallas kernel here")


# speedup vs baseline: 24.4351x; 24.4351x over previous
import functools

import jax
import jax.numpy as jnp
from jax.experimental import pallas as pl

_T = 32
_LEN_M = 8
_BLK = 12800


def _blend_kernel(mpad_ref, x_ref, o_ref):
    mpad = mpad_ref[...]
    t = jax.lax.broadcasted_iota(jnp.int32, (_T, _LEN_M), 0)
    j = jax.lax.broadcasted_iota(jnp.int32, (_T, _LEN_M), 1)
    valid_j = j <= jnp.minimum(t, _LEN_M - 1)
    neg = jnp.float32(-1e30)
    logits = jnp.where(valid_j, mpad, neg)
    logits = logits - jnp.max(logits, axis=1, keepdims=True)
    e = jnp.exp(logits)
    p = e / jnp.sum(e, axis=1, keepdims=True)

    tt = jax.lax.broadcasted_iota(jnp.int32, (_T, _T), 0)
    ss = jax.lax.broadcasted_iota(jnp.int32, (_T, _T), 1)
    start = jnp.maximum(tt - (_LEN_M - 1), 0)
    jidx = ss - start
    band = (ss <= tt) & (jidx >= 0)
    w = jnp.zeros((_T, _T), jnp.float32)
    for k in range(_LEN_M):
        w = w + jnp.where(band & (jidx == k), p[:, k][:, None], 0.0)

    o_ref[...] = jnp.dot(w, x_ref[...], preferred_element_type=jnp.float32)


def _assemble_mpad(M):
    rows = []
    for t in range(_T):
        src = M[t] if t < _LEN_M else M[t - 1]
        row = src[0]
        if row.shape[0] < _LEN_M:
            row = jnp.pad(row, (0, _LEN_M - row.shape[0]))
        rows.append(row)
    return jnp.stack(rows, axis=0)


@functools.partial(jax.jit, static_argnums=())
def kernel(X, M):
    Tn, Nn, Fn = X.shape
    data = X.reshape(Tn, Nn * Fn)
    mpad = _assemble_mpad(M)
    ncols = Nn * Fn
    grid = ncols // _BLK
    out = pl.pallas_call(
        _blend_kernel,
        grid=(grid,),
        in_specs=[
            pl.BlockSpec((_T, _LEN_M), lambda i: (0, 0)),
            pl.BlockSpec((_T, _BLK), lambda i: (0, i)),
        ],
        out_specs=pl.BlockSpec((_T, _BLK), lambda i: (0, i)),
        out_shape=jax.ShapeDtypeStruct((Tn, ncols), jnp.float32),
    )(mpad, data)
    return out.reshape(X.shape)


# speedup vs baseline: 25.4697x; 1.0423x over previous
import functools

import jax
import jax.numpy as jnp
from jax.experimental import pallas as pl

_T = 32
_LEN_M = 8
_BLK = 64000


def _blend_kernel(mpad_ref, x_ref, o_ref):
    mpad = mpad_ref[...]
    t = jax.lax.broadcasted_iota(jnp.int32, (_T, _LEN_M), 0)
    j = jax.lax.broadcasted_iota(jnp.int32, (_T, _LEN_M), 1)
    valid_j = j <= jnp.minimum(t, _LEN_M - 1)
    neg = jnp.float32(-1e30)
    logits = jnp.where(valid_j, mpad, neg)
    logits = logits - jnp.max(logits, axis=1, keepdims=True)
    e = jnp.exp(logits)
    p = e / jnp.sum(e, axis=1, keepdims=True)

    tt = jax.lax.broadcasted_iota(jnp.int32, (_T, _T), 0)
    ss = jax.lax.broadcasted_iota(jnp.int32, (_T, _T), 1)
    start = jnp.maximum(tt - (_LEN_M - 1), 0)
    jidx = ss - start
    band = (ss <= tt) & (jidx >= 0)
    w = jnp.zeros((_T, _T), jnp.float32)
    for k in range(_LEN_M):
        w = w + jnp.where(band & (jidx == k), p[:, k][:, None], 0.0)

    o_ref[...] = jnp.dot(w, x_ref[...], preferred_element_type=jnp.float32)


def _assemble_mpad(M):
    rows = []
    for t in range(_T):
        src = M[t] if t < _LEN_M else M[t - 1]
        row = src[0]
        if row.shape[0] < _LEN_M:
            row = jnp.pad(row, (0, _LEN_M - row.shape[0]))
        rows.append(row)
    return jnp.stack(rows, axis=0)


@functools.partial(jax.jit, static_argnums=())
def kernel(X, M):
    Tn, Nn, Fn = X.shape
    data = X.reshape(Tn, Nn * Fn)
    mpad = _assemble_mpad(M)
    ncols = Nn * Fn
    grid = ncols // _BLK
    out = pl.pallas_call(
        _blend_kernel,
        grid=(grid,),
        in_specs=[
            pl.BlockSpec((_T, _LEN_M), lambda i: (0, 0)),
            pl.BlockSpec((_T, _BLK), lambda i: (0, i)),
        ],
        out_specs=pl.BlockSpec((_T, _BLK), lambda i: (0, i)),
        out_shape=jax.ShapeDtypeStruct((Tn, ncols), jnp.float32),
    )(mpad, data)
    return out.reshape(X.shape)


# speedup vs baseline: 28.2787x; 1.1103x over previous
import functools

import jax
import jax.numpy as jnp
from jax.experimental import pallas as pl

_T = 32
_LEN_M = 8
_BN = 400


def _softmax_p(mpad):
    logits = mpad - jnp.max(mpad, axis=1, keepdims=True)
    e = jnp.exp(logits)
    return e / jnp.sum(e, axis=1, keepdims=True)


def _blend_kernel(mpad_ref, x_ref, o_ref):
    p = _softmax_p(mpad_ref[...])
    t2 = jax.lax.broadcasted_iota(jnp.int32, (_T, _LEN_M), 0)
    j2 = jax.lax.broadcasted_iota(jnp.int32, (_T, _LEN_M), 1)
    start2 = jnp.maximum(t2 - (_LEN_M - 1), 0)
    x = x_ref[...]
    acc = None
    for r in range(_LEN_M):
        need = t2 - r - start2
        c = jnp.sum(jnp.where(j2 == need, p, 0.0), axis=1)
        cb = c.reshape(_T, 1, 1)
        if r == 0:
            acc = cb * x
        else:
            shifted = jnp.concatenate(
                [jnp.zeros((r, x.shape[1], x.shape[2]), jnp.float32), x[: _T - r]],
                axis=0,
            )
            acc = acc + cb * shifted
    o_ref[...] = acc


def _assemble_mpad(M):
    rows = []
    for t in range(_T):
        src = M[t] if t < _LEN_M else M[t - 1]
        row = src[0]
        if row.shape[0] < _LEN_M:
            row = jnp.pad(row, (0, _LEN_M - row.shape[0]),
                          constant_values=-1e30)
        rows.append(row)
    return jnp.stack(rows, axis=0)


@functools.partial(jax.jit, static_argnums=())
def kernel(X, M):
    Tn, Nn, Fn = X.shape
    mpad = _assemble_mpad(M)
    grid = Nn // _BN
    out = pl.pallas_call(
        _blend_kernel,
        grid=(grid,),
        in_specs=[
            pl.BlockSpec((_T, _LEN_M), lambda i: (0, 0)),
            pl.BlockSpec((_T, _BN, Fn), lambda i: (0, i, 0)),
        ],
        out_specs=pl.BlockSpec((_T, _BN, Fn), lambda i: (0, i, 0)),
        out_shape=jax.ShapeDtypeStruct((Tn, Nn, Fn), jnp.float32),
    )(mpad, X)
    return out


# speedup vs baseline: 34.2414x; 1.2109x over previous
import functools

import jax
import jax.numpy as jnp
from jax.experimental import pallas as pl

_T = 32
_LEN_M = 8
_BN = 400


def _softmax_p(mpad):
    logits = mpad - jnp.max(mpad, axis=1, keepdims=True)
    e = jnp.exp(logits)
    return e / jnp.sum(e, axis=1, keepdims=True)


def _blend_kernel(mpad_ref, x_ref, o_ref):
    p = _softmax_p(mpad_ref[...])
    for t in range(_T):
        start = max(0, t - (_LEN_M - 1))
        acc = p[t, 0] * x_ref[start]
        for j in range(1, t - start + 1):
            acc = acc + p[t, j] * x_ref[start + j]
        o_ref[t] = acc


def _assemble_mpad(M):
    rows = []
    for t in range(_T):
        src = M[t] if t < _LEN_M else M[t - 1]
        row = src[0]
        if row.shape[0] < _LEN_M:
            row = jnp.pad(row, (0, _LEN_M - row.shape[0]),
                          constant_values=-1e30)
        rows.append(row)
    return jnp.stack(rows, axis=0)


@functools.partial(jax.jit, static_argnums=())
def kernel(X, M):
    Tn, Nn, Fn = X.shape
    mpad = _assemble_mpad(M)
    grid = Nn // _BN
    out = pl.pallas_call(
        _blend_kernel,
        grid=(grid,),
        in_specs=[
            pl.BlockSpec((_T, _LEN_M), lambda i: (0, 0)),
            pl.BlockSpec((_T, _BN, Fn), lambda i: (0, i, 0)),
        ],
        out_specs=pl.BlockSpec((_T, _BN, Fn), lambda i: (0, i, 0)),
        out_shape=jax.ShapeDtypeStruct((Tn, Nn, Fn), jnp.float32),
    )(mpad, X)
    return out


# speedup vs baseline: 34.8624x; 1.0181x over previous
import functools

import jax
import jax.numpy as jnp
from jax.experimental import pallas as pl

_T = 32
_LEN_M = 8
_BN = 200


def _softmax_p(mpad):
    logits = mpad - jnp.max(mpad, axis=1, keepdims=True)
    e = jnp.exp(logits)
    return e / jnp.sum(e, axis=1, keepdims=True)


def _blend_kernel(mpad_ref, x_ref, o_ref):
    p = _softmax_p(mpad_ref[...])
    for t in range(_T):
        start = max(0, t - (_LEN_M - 1))
        acc = p[t, 0] * x_ref[start]
        for j in range(1, t - start + 1):
            acc = acc + p[t, j] * x_ref[start + j]
        o_ref[t] = acc


def _assemble_mpad(M):
    rows = []
    for t in range(_T):
        src = M[t] if t < _LEN_M else M[t - 1]
        row = src[0]
        if row.shape[0] < _LEN_M:
            row = jnp.pad(row, (0, _LEN_M - row.shape[0]),
                          constant_values=-1e30)
        rows.append(row)
    return jnp.stack(rows, axis=0)


@functools.partial(jax.jit, static_argnums=())
def kernel(X, M):
    Tn, Nn, Fn = X.shape
    mpad = _assemble_mpad(M)
    grid = Nn // _BN
    out = pl.pallas_call(
        _blend_kernel,
        grid=(grid,),
        in_specs=[
            pl.BlockSpec((_T, _LEN_M), lambda i: (0, 0)),
            pl.BlockSpec((_T, _BN, Fn), lambda i: (0, i, 0)),
        ],
        out_specs=pl.BlockSpec((_T, _BN, Fn), lambda i: (0, i, 0)),
        out_shape=jax.ShapeDtypeStruct((Tn, Nn, Fn), jnp.float32),
    )(mpad, X)
    return out
